# baseline (speedup 1.0000x reference)
import functools

import jax
import jax.numpy as jnp
from jax import lax
from jax.experimental import pallas as pl
from jax.experimental.pallas import tpu as pltpu
from jax.experimental.pallas import tpu_sc as plsc

_CHUNK = 128
_NBUF = 4
_INFLIGHT = 3

_BM = 512


def _sc_gather_ht(h_idx, t_idx, entity_emb):
    B = h_idx.shape[0]
    D = entity_emb.shape[1]
    info = plsc.get_sparse_core_info()
    NC, NS = info.num_cores, info.num_subcores
    NW = NC * NS
    b_per_w = B // NW
    n_chunks = b_per_w // _CHUNK

    mesh = plsc.VectorSubcoreMesh(core_axis_name="c", subcore_axis_name="s")
    out_sds = jax.ShapeDtypeStruct((B, D), jnp.float32)

    @functools.partial(
        pl.kernel,
        mesh=mesh,
        out_type=(out_sds, out_sds),
        scratch_types=(
            [pltpu.VMEM((2 * b_per_w,), jnp.int32)]
            + [pltpu.VMEM((_CHUNK, D), jnp.float32) for _ in range(_NBUF)]
            + [pltpu.SemaphoreType.DMA for _ in range(2 * _NBUF)]
        ),
    )
    def k(h_hbm, t_hbm, ent_hbm, h_out, t_out, idx_v, *bufs_and_sems):
        bufs = bufs_and_sems[:_NBUF]
        gsem = bufs_and_sems[_NBUF:2 * _NBUF]
        ssem = bufs_and_sems[2 * _NBUF:]

        wid = lax.axis_index("s") * NC + lax.axis_index("c")
        base = wid * b_per_w

        pltpu.sync_copy(h_hbm.at[pl.ds(base, b_per_w)],
                        idx_v.at[pl.ds(0, b_per_w)])
        pltpu.sync_copy(t_hbm.at[pl.ds(base, b_per_w)],
                        idx_v.at[pl.ds(b_per_w, b_per_w)])

        tasks = []
        for t, out in enumerate((h_out, t_out)):
            for c in range(n_chunks):
                tasks.append((t * b_per_w + c * _CHUNK, out,
                              base + c * _CHUNK))
        T = len(tasks)

        gathers = [None] * T
        stores = [None] * T

        def gather_start(s):
            ioff, _, _ = tasks[s]
            b = s % _NBUF
            gathers[s] = pltpu.async_copy(
                ent_hbm.at[idx_v.at[pl.ds(ioff, _CHUNK)]], bufs[b], gsem[b])

        def store_start(s):
            _, out, obase = tasks[s]
            b = s % _NBUF
            stores[s] = pltpu.async_copy(
                bufs[b], out.at[pl.ds(obase, _CHUNK)], ssem[b])

        for s in range(min(_INFLIGHT, T)):
            gather_start(s)
        for s in range(T):
            if s >= 1:
                stores[s - 1].wait()
            if s + _INFLIGHT < T:
                gather_start(s + _INFLIGHT)
            gathers[s].wait()
            store_start(s)
        stores[T - 1].wait()

    return k(h_idx, t_idx, entity_emb)


def _tc_lookup_r(r_idx, relation_emb):
    B = r_idx.shape[0]
    R, D = relation_emb.shape
    RP = 1024
    G = B // _BM
    rel_p = jnp.pad(relation_emb, ((0, RP - R), (0, 0)))
    idx3 = r_idx.reshape(G, _BM, 1)

    def body(idx_ref, rel_ref, out_ref):
        rel = rel_ref[...]
        hi = rel.astype(jnp.bfloat16)
        lo = (rel - hi.astype(jnp.float32)).astype(jnp.bfloat16)
        ids = idx_ref[0]
        cols = lax.broadcasted_iota(jnp.int32, (_BM, RP), 1)
        onehot = jnp.where(cols == ids, 1.0, 0.0).astype(jnp.bfloat16)
        acc = jnp.dot(onehot, hi, preferred_element_type=jnp.float32)
        acc = acc + jnp.dot(onehot, lo, preferred_element_type=jnp.float32)
        out_ref[...] = acc

    return pl.pallas_call(
        body,
        grid=(G,),
        in_specs=[
            pl.BlockSpec((1, _BM, 1), lambda i: (i, 0, 0)),
            pl.BlockSpec((RP, D), lambda i: (0, 0)),
        ],
        out_specs=pl.BlockSpec((_BM, D), lambda i: (i, 0)),
        out_shape=jax.ShapeDtypeStruct((B, D), jnp.float32),
    )(idx3, rel_p)


def kernel(h_idx, r_idx, t_idx, entity_emb, relation_emb):
    h, t = _sc_gather_ht(h_idx, t_idx, entity_emb)
    r = _tc_lookup_r(r_idx, relation_emb)
    return (h, r, t)


# speedup vs baseline: 1.4174x; 1.4174x over previous
import functools

import jax
import jax.numpy as jnp
from jax import lax
from jax.experimental import pallas as pl
from jax.experimental.pallas import tpu as pltpu
from jax.experimental.pallas import tpu_sc as plsc

_CHUNK = 128
_NBUF = 4
_INFLIGHT = 3


def kernel(h_idx, r_idx, t_idx, entity_emb, relation_emb):
    B = h_idx.shape[0]
    D = entity_emb.shape[1]
    info = plsc.get_sparse_core_info()
    NC, NS = info.num_cores, info.num_subcores
    NW = NC * NS
    b_per_w = B // NW
    n_chunks = b_per_w // _CHUNK

    mesh = plsc.VectorSubcoreMesh(core_axis_name="c", subcore_axis_name="s")
    out_sds = jax.ShapeDtypeStruct((B, D), jnp.float32)

    @functools.partial(
        pl.kernel,
        mesh=mesh,
        out_type=(out_sds, out_sds, out_sds),
        scratch_types=(
            [pltpu.VMEM((3 * b_per_w,), jnp.int32)]
            + [pltpu.VMEM((_CHUNK, D), jnp.float32) for _ in range(_NBUF)]
            + [pltpu.SemaphoreType.DMA for _ in range(2 * _NBUF)]
        ),
    )
    def k(h_hbm, r_hbm, t_hbm, ent_hbm, rel_hbm, h_out, r_out, t_out,
          idx_v, *bufs_and_sems):
        bufs = bufs_and_sems[:_NBUF]
        gsem = bufs_and_sems[_NBUF:2 * _NBUF]
        ssem = bufs_and_sems[2 * _NBUF:]

        wid = lax.axis_index("s") * NC + lax.axis_index("c")
        base = wid * b_per_w

        pltpu.sync_copy(h_hbm.at[pl.ds(base, b_per_w)],
                        idx_v.at[pl.ds(0, b_per_w)])
        pltpu.sync_copy(r_hbm.at[pl.ds(base, b_per_w)],
                        idx_v.at[pl.ds(b_per_w, b_per_w)])
        pltpu.sync_copy(t_hbm.at[pl.ds(base, b_per_w)],
                        idx_v.at[pl.ds(2 * b_per_w, b_per_w)])

        tasks = []
        for t, (table, out) in enumerate(
                ((ent_hbm, h_out), (rel_hbm, r_out), (ent_hbm, t_out))):
            for c in range(n_chunks):
                tasks.append((table, t * b_per_w + c * _CHUNK, out,
                              base + c * _CHUNK))
        T = len(tasks)

        gathers = [None] * T
        stores = [None] * T

        def gather_start(s):
            table, ioff, _, _ = tasks[s]
            b = s % _NBUF
            gathers[s] = pltpu.async_copy(
                table.at[idx_v.at[pl.ds(ioff, _CHUNK)]], bufs[b], gsem[b])

        def store_start(s):
            _, _, out, obase = tasks[s]
            b = s % _NBUF
            stores[s] = pltpu.async_copy(
                bufs[b], out.at[pl.ds(obase, _CHUNK)], ssem[b])

        for s in range(min(_INFLIGHT, T)):
            gather_start(s)
        for s in range(T):
            if s >= 1:
                stores[s - 1].wait()
            if s + _INFLIGHT < T:
                gather_start(s + _INFLIGHT)
            gathers[s].wait()
            store_start(s)
        stores[T - 1].wait()

    return k(h_idx, r_idx, t_idx, entity_emb, relation_emb)
